# baseline (speedup 1.0000x reference)
import functools

import jax
import jax.numpy as jnp
from jax import lax
from jax.experimental import pallas as pl
from jax.experimental.pallas import tpu as pltpu
from jax.experimental.pallas import tpu_sc as plsc

N = 10000
E = 320000
D = 128
H = 128
L = 5
B = 64
P = 5
C = 2

_NS = 16
_EPT = E // _NS
_K = 80
_NIT = _EPT // _K
_BCH = 5
_NBLK = _NIT // _BCH
_AR = 3456
_NPASS = 3
_NP = _AR * _NPASS
_RPT = _AR // _NS


def _mesh():
    return plsc.VectorSubcoreMesh(core_axis_name="c", subcore_axis_name="s",
                                  num_cores=1, num_subcores=_NS)


def _seg_agg(xs, xf, srcs, dsts, srcf, dstf, zeros):
    @functools.partial(
        pl.kernel,
        out_type=[jax.ShapeDtypeStruct((_NP, H), jnp.float32)] * 2,
        mesh=_mesh(),
        scratch_types=[
            pltpu.VMEM((_BCH, _K), jnp.int32),
            pltpu.VMEM((_BCH, _K), jnp.int32),
            pltpu.VMEM((_BCH, _K, H), jnp.float32),
            pltpu.VMEM((_RPT, H), jnp.float32),
            pltpu.VMEM_SHARED((_AR + 8, H), jnp.float32),
            pltpu.SemaphoreType.DMA,
            pltpu.SemaphoreType.DMA,
        ],
    )
    def body(xs_hbm, xf_hbm, srcs_hbm, dsts_hbm, srcf_hbm, dstf_hbm,
             zeros_hbm, outs_hbm, outf_hbm,
             src_v, dst_v, rows_v, slab_v, acc_sh, gsem, ssem):
        sid = lax.axis_index("s")
        rbase = sid * _RPT
        ebase = sid * _EPT

        def run(x_hbm, src_hbm, dst_hbm, out_hbm, half):
            lo = half * _AR
            pltpu.sync_copy(zeros_hbm, slab_v)
            pltpu.sync_copy(slab_v, acc_sh.at[pl.ds(rbase, _RPT)])
            plsc.subcore_barrier()

            def step(i, carry):
                off = ebase + i * (_BCH * _K)
                gds = []
                for b in range(_BCH):
                    pltpu.sync_copy(
                        src_hbm.at[pl.ds(off + b * _K, _K)], src_v.at[b])
                    pltpu.sync_copy(
                        dst_hbm.at[pl.ds(off + b * _K, _K)], dst_v.at[b])
                    gds.append(pltpu.async_copy(
                        x_hbm.at[src_v.at[b]], rows_v.at[b], gsem))
                for b in range(_BCH):
                    for j in range(_K // 16):
                        d = dst_v[b, pl.ds(j * 16, 16)]
                        dl = d - lo
                        inr = (dl >= 0) & (dl < _AR)
                        dst_v[b, pl.ds(j * 16, 16)] = (
                            jnp.where(inr, dl, _AR))
                for b in range(_BCH):
                    gds[b].wait()
                for b in range(_BCH):
                    pltpu.sync_copy(rows_v.at[b], acc_sh.at[dst_v.at[b]],
                                    add=True)
                return carry

            lax.fori_loop(0, _NBLK, step, 0)
            plsc.subcore_barrier()

            pltpu.sync_copy(acc_sh.at[pl.ds(rbase, _RPT)], slab_v)
            pltpu.sync_copy(slab_v, out_hbm.at[pl.ds(lo + rbase, _RPT)])
            plsc.subcore_barrier()

        for _h in range(_NPASS):
            run(xs_hbm, srcs_hbm, dsts_hbm, outs_hbm, _h)
        for _h in range(_NPASS):
            run(xf_hbm, srcf_hbm, dstf_hbm, outf_hbm, _h)

    return body(xs, xf, srcs, dsts, srcf, dstf, zeros)


_BLK = 1000


def _dense_body(x_ref, agg_ref, w1_ref, b1_ref, w2_ref,
                b2_ref, eps_ref, out_ref):
    z = (1.0 + eps_ref[0, 0]) * x_ref[...] + agg_ref[...]
    z = jnp.maximum(
        jnp.dot(z, w1_ref[...], preferred_element_type=jnp.float32)
        + b1_ref[...], 0.0)
    out_ref[...] = jnp.maximum(
        jnp.dot(z, w2_ref[...], preferred_element_type=jnp.float32)
        + b2_ref[...], 0.0)


def _dense_layer(x, agg, w1, b1, w2, b2, eps):
    return pl.pallas_call(
        _dense_body,
        grid=(N // _BLK,),
        in_specs=[
            pl.BlockSpec((_BLK, H), lambda i: (i, 0)),
            pl.BlockSpec((_BLK, H), lambda i: (i, 0)),
            pl.BlockSpec((H, H), lambda i: (0, 0)),
            pl.BlockSpec((1, H), lambda i: (0, 0)),
            pl.BlockSpec((H, H), lambda i: (0, 0)),
            pl.BlockSpec((1, H), lambda i: (0, 0)),
            pl.BlockSpec((1, 1), lambda i: (0, 0)),
        ],
        out_specs=pl.BlockSpec((_BLK, H), lambda i: (i, 0)),
        out_shape=jax.ShapeDtypeStruct((N, H), jnp.float32),
    )(x, agg, w1, b1, w2, b2, eps)


_PC = 2000
_NCH = N // _PC


def _head_body(hs_ref, hf_ref, sb_ref, fb_ref,
               sp_prompts_ref, fp_prompts_ref, aW1_ref, ab1_ref, aW2_ref,
               ab2_ref,
               gW1s_ref, gb1_ref, gW2_ref, gb2_ref, pgW_ref, pgb_ref,
               fuW1a_ref, fuW1b_ref, fub1_ref, fuW2_ref, fub2_ref,
               clsW_ref, clsb_ref,
               logits_ref, ortho_ref, sums, cnts):
    i = pl.program_id(0)

    @pl.when(i == 0)
    def _():
        sums[...] = jnp.zeros_like(sums)
        cnts[...] = jnp.zeros_like(cnts)

    for s, bref, href in ((0, sb_ref, hs_ref), (1, fb_ref, hf_ref)):
        bv = bref[0, 0, :]
        oh = (bv[None, :] == lax.broadcasted_iota(jnp.int32, (B, _PC), 0)
              ).astype(jnp.float32)
        sums[s] += jnp.dot(oh, href[...], preferred_element_type=jnp.float32)
        cnts[s] += jnp.broadcast_to(
            jnp.sum(oh, axis=1, keepdims=True), (B, H))

    @pl.when(i == _NCH - 1)
    def _():
        sf = sums[0] / jnp.maximum(cnts[0], 1.0)
        ff = sums[1] / jnp.maximum(cnts[1], 1.0)
        a = jnp.maximum(
            jnp.dot(sf, aW1_ref[...], preferred_element_type=jnp.float32)
            + ab1_ref[...], 0.0)
        wts = jax.nn.softmax(
            jnp.dot(a, aW2_ref[...], preferred_element_type=jnp.float32)
            + ab2_ref[...], axis=-1)
        sf = sf + jnp.dot(wts, sp_prompts_ref[...],
                          preferred_element_type=jnp.float32)
        dyn = jnp.dot(ff, pgW_ref[...],
                      preferred_element_type=jnp.float32) + pgb_ref[...]
        static = jnp.broadcast_to(
            jnp.mean(fp_prompts_ref[...], axis=0)[None, :], (B, H))
        gz = jnp.maximum(
            jnp.dot(ff, gW1s_ref[...], preferred_element_type=jnp.float32)
            + gb1_ref[...], 0.0)
        g = jax.nn.sigmoid(
            jnp.dot(gz, gW2_ref[...], preferred_element_type=jnp.float32)
            + gb2_ref[...])
        ff = ff + g * dyn + (1.0 - g) * static
        eps_n = 1e-08
        n1 = sf / jnp.maximum(
            jnp.sqrt(jnp.sum(sf * sf, axis=1, keepdims=True)), eps_n)
        n2 = ff / jnp.maximum(
            jnp.sqrt(jnp.sum(ff * ff, axis=1, keepdims=True)), eps_n)
        sim = jnp.dot(n1, n2.T, preferred_element_type=jnp.float32)
        ortho_ref[...] = (jnp.mean(jnp.abs(sim)) * 0.1).reshape(1, 1)
        fz = jnp.maximum(
            jnp.dot(sf, fuW1a_ref[...], preferred_element_type=jnp.float32)
            + jnp.dot(ff, fuW1b_ref[...], preferred_element_type=jnp.float32)
            + fub1_ref[...], 0.0)
        fused = jnp.dot(fz, fuW2_ref[...],
                        preferred_element_type=jnp.float32) + fub2_ref[...]
        logits_ref[...] = jnp.dot(
            fused, clsW_ref[...],
            preferred_element_type=jnp.float32) + clsb_ref[...]


def _head(hs, hf, sb_r, fb_r, sp_prompts, fp_prompts, aW1, ab1, aW2, ab2,
          gW1s, gb1, gW2, gb2, pgW, pgb, fuW1a, fuW1b, fub1, fuW2, fub2,
          clsW, clsb):
    full = lambda shape: pl.BlockSpec(shape, lambda i: tuple(0 for _ in shape))
    return pl.pallas_call(
        _head_body,
        grid=(_NCH,),
        in_specs=[
            pl.BlockSpec((_PC, H), lambda i: (i, 0)),
            pl.BlockSpec((_PC, H), lambda i: (i, 0)),
            pl.BlockSpec((1, 1, _PC), lambda i: (i, 0, 0)),
            pl.BlockSpec((1, 1, _PC), lambda i: (i, 0, 0)),
            full((P, H)),
            full((P, H)),
            full((H, H // 2)),
            full((1, H // 2)),
            full((H // 2, P)),
            full((1, P)),
            full((H, H)),
            full((1, H)),
            full((H, 1)),
            full((1, 1)),
            full((H, H)),
            full((1, H)),
            full((H, H)),
            full((H, H)),
            full((1, H)),
            full((H, H)),
            full((1, H)),
            full((H, C)),
            full((1, C)),
        ],
        out_specs=[
            pl.BlockSpec((B, C), lambda i: (0, 0)),
            pl.BlockSpec((1, 1), lambda i: (0, 0)),
        ],
        out_shape=[
            jax.ShapeDtypeStruct((B, C), jnp.float32),
            jax.ShapeDtypeStruct((1, 1), jnp.float32),
        ],
        scratch_shapes=[
            pltpu.VMEM((2, B, H), jnp.float32),
            pltpu.VMEM((2, B, H), jnp.float32),
        ],
    )(hs, hf, sb_r, fb_r, sp_prompts, fp_prompts, aW1, ab1, aW2, ab2,
      gW1s, gb1, gW2, gb2, pgW, pgb, fuW1a, fuW1b, fub1, fuW2, fub2,
      clsW, clsb)


def kernel(struct_x, func_x, struct_edge_index, func_edge_index,
           struct_batch, func_batch,
           sW1, sb1, sW2, sb2, sEps, fW1, fb1, fW2, fb2, fEps,
           sp_prompts, sp_aW1, sp_ab1, sp_aW2, sp_ab2,
           fp_prompts, fp_gW1, fp_gb1, fp_gW2, fp_gb2, fp_pgW, fp_pgb,
           fu_W1, fu_b1, fu_W2, fu_b2, cls_W, cls_b):
    srcs, dsts = struct_edge_index[0], struct_edge_index[1]
    srcf, dstf = func_edge_index[0], func_edge_index[1]
    zeros = jnp.zeros((_RPT, H), jnp.float32)

    def layer_step(l, carry):
        hs, hf = carry
        pick = lambda a: lax.dynamic_index_in_dim(a, l, 0, keepdims=False)
        agg_s, agg_f = _seg_agg(hs, hf, srcs, dsts, srcf, dstf, zeros)
        hs = _dense_layer(hs, agg_s, pick(sW1), pick(sb1)[None, :],
                          pick(sW2), pick(sb2)[None, :],
                          pick(sEps)[None, None])
        hf = _dense_layer(hf, agg_f, pick(fW1), pick(fb1)[None, :],
                          pick(fW2), pick(fb2)[None, :],
                          pick(fEps)[None, None])
        return (hs, hf)

    a0 = jnp.abs(srcs[0])
    trip = L + a0 // (a0 * a0 + 1)
    hs, hf = lax.fori_loop(0, trip, layer_step, (struct_x, func_x))

    sb_r = struct_batch.reshape(_NCH, 1, _PC)
    fb_r = func_batch.reshape(_NCH, 1, _PC)
    gW1s = fp_gW1[:H] + fp_gW1[H:]
    fuW1a, fuW1b = fu_W1[:H], fu_W1[H:]

    logits, ortho = _head(
        hs, hf, sb_r, fb_r, sp_prompts, fp_prompts,
        sp_aW1, sp_ab1[None, :], sp_aW2, sp_ab2[None, :],
        gW1s, fp_gb1[None, :], fp_gW2, fp_gb2[None, :],
        fp_pgW, fp_pgb[None, :],
        fuW1a, fuW1b, fu_b1[None, :], fu_W2, fu_b2[None, :],
        cls_W, cls_b[None, :])
    return logits, ortho.reshape(())


# speedup vs baseline: 1.0273x; 1.0273x over previous
import functools

import jax
import jax.numpy as jnp
from jax import lax
from jax.experimental import pallas as pl
from jax.experimental.pallas import tpu as pltpu
from jax.experimental.pallas import tpu_sc as plsc

N = 10000
E = 320000
D = 128
H = 128
L = 5
B = 64
P = 5
C = 2

_NS = 16
_EPT = E // _NS
_K = 80
_NIT = _EPT // _K
_BCH = 5
_NBLK = _NIT // _BCH
_AR = 3456
_NPASS = 3
_NP = _AR * _NPASS
_RPT = _AR // _NS


def _mesh():
    return plsc.VectorSubcoreMesh(core_axis_name="c", subcore_axis_name="s",
                                  num_cores=1, num_subcores=_NS)


def _seg_agg(xs, xf, srcs, dsts, srcf, dstf, zeros):
    @functools.partial(
        pl.kernel,
        out_type=[jax.ShapeDtypeStruct((_NP, H), jnp.float32)] * 2,
        mesh=_mesh(),
        scratch_types=[
            pltpu.VMEM((_BCH, _K), jnp.int32),
            pltpu.VMEM((_BCH, _K), jnp.int32),
            pltpu.VMEM((_BCH, _K, H), jnp.float32),
            pltpu.VMEM((_RPT, H), jnp.float32),
            pltpu.VMEM_SHARED((_AR + 8, H), jnp.float32),
            pltpu.SemaphoreType.DMA,
            pltpu.SemaphoreType.DMA,
        ],
    )
    def body(xs_hbm, xf_hbm, srcs_hbm, dsts_hbm, srcf_hbm, dstf_hbm,
             zeros_hbm, outs_hbm, outf_hbm,
             src_v, dst_v, rows_v, slab_v, acc_sh, gsem, ssem):
        sid = lax.axis_index("s")
        rbase = sid * _RPT
        ebase = sid * _EPT

        def run(x_hbm, src_hbm, dst_hbm, out_hbm, half):
            lo = half * _AR
            pltpu.sync_copy(zeros_hbm, slab_v)
            pltpu.sync_copy(slab_v, acc_sh.at[pl.ds(rbase, _RPT)])
            plsc.subcore_barrier()

            def step(i, carry):
                off = ebase + i * (_BCH * _K)
                gds = []
                for b in range(_BCH):
                    pltpu.sync_copy(
                        src_hbm.at[pl.ds(off + b * _K, _K)], src_v.at[b])
                    pltpu.sync_copy(
                        dst_hbm.at[pl.ds(off + b * _K, _K)], dst_v.at[b])
                    gds.append(pltpu.async_copy(
                        x_hbm.at[src_v.at[b]], rows_v.at[b], gsem))
                for b in range(_BCH):
                    for j in range(_K // 16):
                        d = dst_v[b, pl.ds(j * 16, 16)]
                        dl = d - lo
                        inr = (dl >= 0) & (dl < _AR)
                        dst_v[b, pl.ds(j * 16, 16)] = (
                            jnp.where(inr, dl, _AR))
                for b in range(_BCH):
                    gds[b].wait()
                sds = [pltpu.async_copy(rows_v.at[b],
                                        acc_sh.at[dst_v.at[b]],
                                        gsem, add=True)
                       for b in range(_BCH)]
                for b in range(_BCH):
                    sds[b].wait()
                return carry

            lax.fori_loop(0, _NBLK, step, 0)
            plsc.subcore_barrier()

            pltpu.sync_copy(acc_sh.at[pl.ds(rbase, _RPT)], slab_v)
            pltpu.sync_copy(slab_v, out_hbm.at[pl.ds(lo + rbase, _RPT)])
            plsc.subcore_barrier()

        for _h in range(_NPASS):
            run(xs_hbm, srcs_hbm, dsts_hbm, outs_hbm, _h)
        for _h in range(_NPASS):
            run(xf_hbm, srcf_hbm, dstf_hbm, outf_hbm, _h)

    return body(xs, xf, srcs, dsts, srcf, dstf, zeros)


_BLK = 1000


def _dense_body(x_ref, agg_ref, w1_ref, b1_ref, w2_ref,
                b2_ref, eps_ref, out_ref):
    z = (1.0 + eps_ref[0, 0]) * x_ref[...] + agg_ref[...]
    z = jnp.maximum(
        jnp.dot(z, w1_ref[...], preferred_element_type=jnp.float32)
        + b1_ref[...], 0.0)
    out_ref[...] = jnp.maximum(
        jnp.dot(z, w2_ref[...], preferred_element_type=jnp.float32)
        + b2_ref[...], 0.0)


def _dense_layer(x, agg, w1, b1, w2, b2, eps):
    return pl.pallas_call(
        _dense_body,
        grid=(N // _BLK,),
        in_specs=[
            pl.BlockSpec((_BLK, H), lambda i: (i, 0)),
            pl.BlockSpec((_BLK, H), lambda i: (i, 0)),
            pl.BlockSpec((H, H), lambda i: (0, 0)),
            pl.BlockSpec((1, H), lambda i: (0, 0)),
            pl.BlockSpec((H, H), lambda i: (0, 0)),
            pl.BlockSpec((1, H), lambda i: (0, 0)),
            pl.BlockSpec((1, 1), lambda i: (0, 0)),
        ],
        out_specs=pl.BlockSpec((_BLK, H), lambda i: (i, 0)),
        out_shape=jax.ShapeDtypeStruct((N, H), jnp.float32),
    )(x, agg, w1, b1, w2, b2, eps)


_PC = 2000
_NCH = N // _PC


def _head_body(hs_ref, hf_ref, sb_ref, fb_ref,
               sp_prompts_ref, fp_prompts_ref, aW1_ref, ab1_ref, aW2_ref,
               ab2_ref,
               gW1s_ref, gb1_ref, gW2_ref, gb2_ref, pgW_ref, pgb_ref,
               fuW1a_ref, fuW1b_ref, fub1_ref, fuW2_ref, fub2_ref,
               clsW_ref, clsb_ref,
               logits_ref, ortho_ref, sums, cnts):
    i = pl.program_id(0)

    @pl.when(i == 0)
    def _():
        sums[...] = jnp.zeros_like(sums)
        cnts[...] = jnp.zeros_like(cnts)

    for s, bref, href in ((0, sb_ref, hs_ref), (1, fb_ref, hf_ref)):
        bv = bref[0, 0, :]
        oh = (bv[None, :] == lax.broadcasted_iota(jnp.int32, (B, _PC), 0)
              ).astype(jnp.float32)
        sums[s] += jnp.dot(oh, href[...], preferred_element_type=jnp.float32)
        cnts[s] += jnp.broadcast_to(
            jnp.sum(oh, axis=1, keepdims=True), (B, H))

    @pl.when(i == _NCH - 1)
    def _():
        sf = sums[0] / jnp.maximum(cnts[0], 1.0)
        ff = sums[1] / jnp.maximum(cnts[1], 1.0)
        a = jnp.maximum(
            jnp.dot(sf, aW1_ref[...], preferred_element_type=jnp.float32)
            + ab1_ref[...], 0.0)
        wts = jax.nn.softmax(
            jnp.dot(a, aW2_ref[...], preferred_element_type=jnp.float32)
            + ab2_ref[...], axis=-1)
        sf = sf + jnp.dot(wts, sp_prompts_ref[...],
                          preferred_element_type=jnp.float32)
        dyn = jnp.dot(ff, pgW_ref[...],
                      preferred_element_type=jnp.float32) + pgb_ref[...]
        static = jnp.broadcast_to(
            jnp.mean(fp_prompts_ref[...], axis=0)[None, :], (B, H))
        gz = jnp.maximum(
            jnp.dot(ff, gW1s_ref[...], preferred_element_type=jnp.float32)
            + gb1_ref[...], 0.0)
        g = jax.nn.sigmoid(
            jnp.dot(gz, gW2_ref[...], preferred_element_type=jnp.float32)
            + gb2_ref[...])
        ff = ff + g * dyn + (1.0 - g) * static
        eps_n = 1e-08
        n1 = sf / jnp.maximum(
            jnp.sqrt(jnp.sum(sf * sf, axis=1, keepdims=True)), eps_n)
        n2 = ff / jnp.maximum(
            jnp.sqrt(jnp.sum(ff * ff, axis=1, keepdims=True)), eps_n)
        sim = jnp.dot(n1, n2.T, preferred_element_type=jnp.float32)
        ortho_ref[...] = (jnp.mean(jnp.abs(sim)) * 0.1).reshape(1, 1)
        fz = jnp.maximum(
            jnp.dot(sf, fuW1a_ref[...], preferred_element_type=jnp.float32)
            + jnp.dot(ff, fuW1b_ref[...], preferred_element_type=jnp.float32)
            + fub1_ref[...], 0.0)
        fused = jnp.dot(fz, fuW2_ref[...],
                        preferred_element_type=jnp.float32) + fub2_ref[...]
        logits_ref[...] = jnp.dot(
            fused, clsW_ref[...],
            preferred_element_type=jnp.float32) + clsb_ref[...]


def _head(hs, hf, sb_r, fb_r, sp_prompts, fp_prompts, aW1, ab1, aW2, ab2,
          gW1s, gb1, gW2, gb2, pgW, pgb, fuW1a, fuW1b, fub1, fuW2, fub2,
          clsW, clsb):
    full = lambda shape: pl.BlockSpec(shape, lambda i: tuple(0 for _ in shape))
    return pl.pallas_call(
        _head_body,
        grid=(_NCH,),
        in_specs=[
            pl.BlockSpec((_PC, H), lambda i: (i, 0)),
            pl.BlockSpec((_PC, H), lambda i: (i, 0)),
            pl.BlockSpec((1, 1, _PC), lambda i: (i, 0, 0)),
            pl.BlockSpec((1, 1, _PC), lambda i: (i, 0, 0)),
            full((P, H)),
            full((P, H)),
            full((H, H // 2)),
            full((1, H // 2)),
            full((H // 2, P)),
            full((1, P)),
            full((H, H)),
            full((1, H)),
            full((H, 1)),
            full((1, 1)),
            full((H, H)),
            full((1, H)),
            full((H, H)),
            full((H, H)),
            full((1, H)),
            full((H, H)),
            full((1, H)),
            full((H, C)),
            full((1, C)),
        ],
        out_specs=[
            pl.BlockSpec((B, C), lambda i: (0, 0)),
            pl.BlockSpec((1, 1), lambda i: (0, 0)),
        ],
        out_shape=[
            jax.ShapeDtypeStruct((B, C), jnp.float32),
            jax.ShapeDtypeStruct((1, 1), jnp.float32),
        ],
        scratch_shapes=[
            pltpu.VMEM((2, B, H), jnp.float32),
            pltpu.VMEM((2, B, H), jnp.float32),
        ],
    )(hs, hf, sb_r, fb_r, sp_prompts, fp_prompts, aW1, ab1, aW2, ab2,
      gW1s, gb1, gW2, gb2, pgW, pgb, fuW1a, fuW1b, fub1, fuW2, fub2,
      clsW, clsb)


def kernel(struct_x, func_x, struct_edge_index, func_edge_index,
           struct_batch, func_batch,
           sW1, sb1, sW2, sb2, sEps, fW1, fb1, fW2, fb2, fEps,
           sp_prompts, sp_aW1, sp_ab1, sp_aW2, sp_ab2,
           fp_prompts, fp_gW1, fp_gb1, fp_gW2, fp_gb2, fp_pgW, fp_pgb,
           fu_W1, fu_b1, fu_W2, fu_b2, cls_W, cls_b):
    srcs, dsts = struct_edge_index[0], struct_edge_index[1]
    srcf, dstf = func_edge_index[0], func_edge_index[1]
    zeros = jnp.zeros((_RPT, H), jnp.float32)

    def layer_step(l, carry):
        hs, hf = carry
        pick = lambda a: lax.dynamic_index_in_dim(a, l, 0, keepdims=False)
        agg_s, agg_f = _seg_agg(hs, hf, srcs, dsts, srcf, dstf, zeros)
        hs = _dense_layer(hs, agg_s, pick(sW1), pick(sb1)[None, :],
                          pick(sW2), pick(sb2)[None, :],
                          pick(sEps)[None, None])
        hf = _dense_layer(hf, agg_f, pick(fW1), pick(fb1)[None, :],
                          pick(fW2), pick(fb2)[None, :],
                          pick(fEps)[None, None])
        return (hs, hf)

    a0 = jnp.abs(srcs[0])
    trip = L + a0 // (a0 * a0 + 1)
    hs, hf = lax.fori_loop(0, trip, layer_step, (struct_x, func_x))

    sb_r = struct_batch.reshape(_NCH, 1, _PC)
    fb_r = func_batch.reshape(_NCH, 1, _PC)
    gW1s = fp_gW1[:H] + fp_gW1[H:]
    fuW1a, fuW1b = fu_W1[:H], fu_W1[H:]

    logits, ortho = _head(
        hs, hf, sb_r, fb_r, sp_prompts, fp_prompts,
        sp_aW1, sp_ab1[None, :], sp_aW2, sp_ab2[None, :],
        gW1s, fp_gb1[None, :], fp_gW2, fp_gb2[None, :],
        fp_pgW, fp_pgb[None, :],
        fuW1a, fuW1b, fu_b1[None, :], fu_W2, fu_b2[None, :],
        cls_W, cls_b[None, :])
    return logits, ortho.reshape(())
